# baseline (speedup 1.0000x reference)
import functools

import jax
import jax.numpy as jnp
from jax import lax
from jax.experimental import pallas as pl
from jax.experimental.pallas import tpu as pltpu
from jax.experimental.pallas import tpu_sc as plsc

_GROUPS = 4
_LANES = 16
_NC, _NS = 2, 16
_NW = _NC * _NS


def _sc_maxrel(xT, idx_j, idx_i, M, C, K):
    E = M * K
    EC = 128
    NPC = EC // K
    NCH = E // EC
    T = -(-NCH // _NW)
    T += T % 2
    NCHP = _NW * T
    Mp = NCHP * NPC

    ij2 = jnp.zeros((NCHP, EC), jnp.int32).at[:NCH].set(idx_j.reshape(NCH, EC))
    ii2 = jnp.zeros((NCHP, EC), jnp.int32).at[:NCH].set(idx_i.reshape(NCH, EC))

    mesh = plsc.VectorSubcoreMesh(core_axis_name="c", subcore_axis_name="s")

    @functools.partial(
        pl.kernel,
        mesh=mesh,
        out_type=jax.ShapeDtypeStruct((Mp, C), jnp.float32),
        scratch_types=[
            pltpu.VMEM((EC,), jnp.int32),
            pltpu.VMEM((EC,), jnp.int32),
            pltpu.VMEM((EC,), jnp.int32),
            pltpu.VMEM((EC,), jnp.int32),
            pltpu.VMEM((EC, C), jnp.float32),
            pltpu.VMEM((EC, C), jnp.float32),
            pltpu.VMEM((EC, C), jnp.float32),
            pltpu.VMEM((EC, C), jnp.float32),
            pltpu.VMEM((NPC, C), jnp.float32),
            pltpu.VMEM((NPC, C), jnp.float32),
            pltpu.SemaphoreType.DMA,
            pltpu.SemaphoreType.DMA,
            pltpu.SemaphoreType.DMA,
            pltpu.SemaphoreType.DMA,
            pltpu.SemaphoreType.DMA,
            pltpu.SemaphoreType.DMA,
        ],
    )
    def sc_kernel(xT_hbm, ij_hbm, ii_hbm, out_hbm,
                  ij0, ij1, ii0, ii1, rj0, rj1, ri0, ri1, ov0, ov1,
                  sj0, sj1, si0, si1, so0, so1):
        ij_v = (ij0, ij1)
        ii_v = (ii0, ii1)
        rj_v = (rj0, rj1)
        ri_v = (ri0, ri1)
        o_v = (ov0, ov1)
        semj = (sj0, sj1)
        semi = (si0, si1)
        semo = (so0, so1)
        wid = lax.axis_index("s") * _NC + lax.axis_index("c")

        def fetch(t, b):
            ch = wid + t * _NW
            pltpu.sync_copy(ij_hbm.at[ch], ij_v[b])
            pltpu.sync_copy(ii_hbm.at[ch], ii_v[b])
            pltpu.async_copy(xT_hbm.at[ij_v[b]], rj_v[b], semj[b])
            pltpu.async_copy(xT_hbm.at[ii_v[b]], ri_v[b], semi[b])

        for b in range(2):
            fetch(b, b)

        def compute(b):
            def node(n, c2):
                for cc in range(C // _LANES):
                    sl = pl.ds(cc * _LANES, _LANES)
                    acc = rj_v[b][n * K, sl] - ri_v[b][n * K, sl]
                    for kk in range(1, K):
                        acc = jnp.maximum(
                            acc, rj_v[b][n * K + kk, sl] - ri_v[b][n * K + kk, sl])
                    o_v[b][n, sl] = acc
                return c2
            lax.fori_loop(0, NPC, node, 0)

        def wait_gathers(b):
            pltpu.make_async_copy(xT_hbm.at[ij_v[b]], rj_v[b], semj[b]).wait()
            pltpu.make_async_copy(xT_hbm.at[ii_v[b]], ri_v[b], semi[b]).wait()

        def writeback(t, b):
            r0 = (wid + t * _NW) * NPC
            pltpu.sync_copy(o_v[b], out_hbm.at[pl.ds(r0, NPC)])

        for b in range(2):
            wait_gathers(b)
            compute(b)
            writeback(b, b)
            fetch(b + 2, b)

        def outer(g, carry):
            for b in range(2):
                t = g + b
                wait_gathers(b)
                compute(b)
                writeback(t, b)
                fetch(t + 2, b)
            return carry

        lax.fori_loop(0, (T - 4) // 2, lambda i, c: outer(2 + i * 2, c), 0)

        for b in range(2):
            wait_gathers(b)
            compute(b)
            writeback(T - 2 + b, b)

    out = sc_kernel(xT, ij2, ii2)
    return out[:M]


def _tc_body(wx_ref, wj_ref, b_ref, x_ref, mr_ref, o_ref):
    xb = x_ref[0]
    mr = mr_ref[0]
    acc = jnp.dot(wx_ref[...], xb, preferred_element_type=jnp.float32)
    acc = acc + lax.dot_general(
        wj_ref[...], mr, (((1,), (1,)), ((), ())),
        preferred_element_type=jnp.float32)
    o_ref[0] = jnp.maximum(acc + b_ref[...], 0.0)


def _tc_conv(x3, mr3, Wx, Wj, bias):
    B, C, N = x3.shape
    COUT = Wx.shape[0]
    grid = (B,)
    return pl.pallas_call(
        _tc_body,
        grid=grid,
        in_specs=[
            pl.BlockSpec((COUT, C), lambda b: (0, 0)),
            pl.BlockSpec((COUT, C), lambda b: (0, 0)),
            pl.BlockSpec((COUT, 1), lambda b: (0, 0)),
            pl.BlockSpec((1, C, N), lambda b: (b, 0, 0)),
            pl.BlockSpec((1, N, C), lambda b: (b, 0, 0)),
        ],
        out_specs=pl.BlockSpec((1, COUT, N), lambda b: (b, 0, 0)),
        out_shape=jax.ShapeDtypeStruct((B, COUT, N), jnp.float32),
    )(Wx, Wj, bias.reshape(COUT, 1), x3, mr3)


def _block_diag(blocks):
    G, R, S = blocks.shape
    out = jnp.zeros((G * R, G * S), blocks.dtype)
    for g in range(G):
        out = out.at[g * R:(g + 1) * R, g * S:(g + 1) * S].set(blocks[g])
    return out


def kernel(x, edge_index, W, bias):
    B, C, N, _ = x.shape
    K = edge_index.shape[-1]
    COUT = W.shape[0]

    x3 = x[..., 0]
    xT = jnp.transpose(x3, (0, 2, 1)).reshape(B * N, C)
    ei = edge_index.astype(jnp.int32)
    base = (jnp.arange(B, dtype=jnp.int32) * N)[:, None, None]
    idx_j = (ei[0] + base).reshape(B * N * K)
    idx_i = (ei[1] + base).reshape(B * N * K)

    mr = _sc_maxrel(xT, idx_j, idx_i, B * N, C, K)

    Wg = W[:, :, 0, 0].reshape(_GROUPS, COUT // _GROUPS, (2 * C) // _GROUPS)
    Wx = _block_diag(Wg[:, :, 0::2])
    Wj = _block_diag(Wg[:, :, 1::2])

    out = _tc_conv(x3, mr.reshape(B, N, C), Wx, Wj, bias)
    return out[..., None]


# speedup vs baseline: 1.0023x; 1.0023x over previous
import functools

import jax
import jax.numpy as jnp
from jax import lax
from jax.experimental import pallas as pl
from jax.experimental.pallas import tpu as pltpu
from jax.experimental.pallas import tpu_sc as plsc

_GROUPS = 4
_LANES = 16
_NC, _NS = 2, 16
_NW = _NC * _NS


def _sc_maxrel(xT, idx_j, idx_i, M, C, K):
    E = M * K
    EC = 128
    NPC = EC // K
    NCH = E // EC
    T = -(-NCH // _NW)
    T += T % 2
    NCHP = _NW * T
    Mp = NCHP * NPC

    ij2 = jnp.zeros((NCHP, EC), jnp.int32).at[:NCH].set(idx_j.reshape(NCH, EC))
    ii2 = jnp.zeros((NCHP, EC), jnp.int32).at[:NCH].set(idx_i.reshape(NCH, EC))

    mesh = plsc.VectorSubcoreMesh(core_axis_name="c", subcore_axis_name="s")

    @functools.partial(
        pl.kernel,
        mesh=mesh,
        out_type=jax.ShapeDtypeStruct((Mp, C), jnp.float32),
        scratch_types=[
            pltpu.VMEM((EC,), jnp.int32),
            pltpu.VMEM((EC,), jnp.int32),
            pltpu.VMEM((EC,), jnp.int32),
            pltpu.VMEM((EC,), jnp.int32),
            pltpu.VMEM((EC, C), jnp.float32),
            pltpu.VMEM((EC, C), jnp.float32),
            pltpu.VMEM((EC, C), jnp.float32),
            pltpu.VMEM((EC, C), jnp.float32),
            pltpu.VMEM((NPC, C), jnp.float32),
            pltpu.VMEM((NPC, C), jnp.float32),
            pltpu.SemaphoreType.DMA,
            pltpu.SemaphoreType.DMA,
            pltpu.SemaphoreType.DMA,
            pltpu.SemaphoreType.DMA,
            pltpu.SemaphoreType.DMA,
            pltpu.SemaphoreType.DMA,
        ],
    )
    def sc_kernel(xT_hbm, ij_hbm, ii_hbm, out_hbm,
                  ij0, ij1, ii0, ii1, rj0, rj1, ri0, ri1, ov0, ov1,
                  sj0, sj1, si0, si1, so0, so1):
        ij_v = (ij0, ij1)
        ii_v = (ii0, ii1)
        rj_v = (rj0, rj1)
        ri_v = (ri0, ri1)
        o_v = (ov0, ov1)
        semj = (sj0, sj1)
        semi = (si0, si1)
        semo = (so0, so1)
        wid = lax.axis_index("s") * _NC + lax.axis_index("c")

        def fetch(t, b):
            ch = wid + t * _NW
            pltpu.sync_copy(ij_hbm.at[ch], ij_v[b])
            pltpu.sync_copy(ii_hbm.at[ch], ii_v[b])
            pltpu.async_copy(xT_hbm.at[ij_v[b]], rj_v[b], semj[b])
            pltpu.async_copy(xT_hbm.at[ii_v[b]], ri_v[b], semi[b])

        for b in range(2):
            fetch(b, b)

        def compute(b):
            def node(n, c2):
                for cc in range(C // _LANES):
                    sl = pl.ds(cc * _LANES, _LANES)
                    acc = rj_v[b][n * K, sl] - ri_v[b][n * K, sl]
                    for kk in range(1, K):
                        acc = jnp.maximum(
                            acc, rj_v[b][n * K + kk, sl] - ri_v[b][n * K + kk, sl])
                    o_v[b][n, sl] = acc
                return c2
            lax.fori_loop(0, NPC, node, 0)

        def wait_gathers(b):
            pltpu.make_async_copy(xT_hbm.at[ij_v[b]], rj_v[b], semj[b]).wait()
            pltpu.make_async_copy(xT_hbm.at[ii_v[b]], ri_v[b], semi[b]).wait()

        def writeback(t, b):
            r0 = (wid + t * _NW) * NPC
            pltpu.sync_copy(o_v[b], out_hbm.at[pl.ds(r0, NPC)])

        for b in range(2):
            wait_gathers(b)
            compute(b)
            writeback(b, b)
            fetch(b + 2, b)

        def outer(g, carry):
            for b in range(2):
                t = g + b
                with jax.named_scope("waitg"):
                    wait_gathers(b)
                with jax.named_scope("comp"):
                    compute(b)
                with jax.named_scope("wb"):
                    writeback(t, b)
                with jax.named_scope("fetch"):
                    fetch(t + 2, b)
            return carry

        lax.fori_loop(0, (T - 4) // 2, lambda i, c: outer(2 + i * 2, c), 0)

        for b in range(2):
            wait_gathers(b)
            compute(b)
            writeback(T - 2 + b, b)

    out = sc_kernel(xT, ij2, ii2)
    return out[:M]


def _tc_body(wx_ref, wj_ref, b_ref, x_ref, mr_ref, o_ref):
    xb = x_ref[0]
    mr = mr_ref[0]
    acc = jnp.dot(wx_ref[...], xb, preferred_element_type=jnp.float32)
    acc = acc + lax.dot_general(
        wj_ref[...], mr, (((1,), (1,)), ((), ())),
        preferred_element_type=jnp.float32)
    o_ref[0] = jnp.maximum(acc + b_ref[...], 0.0)


def _tc_conv(x3, mr3, Wx, Wj, bias):
    B, C, N = x3.shape
    COUT = Wx.shape[0]
    grid = (B,)
    return pl.pallas_call(
        _tc_body,
        grid=grid,
        in_specs=[
            pl.BlockSpec((COUT, C), lambda b: (0, 0)),
            pl.BlockSpec((COUT, C), lambda b: (0, 0)),
            pl.BlockSpec((COUT, 1), lambda b: (0, 0)),
            pl.BlockSpec((1, C, N), lambda b: (b, 0, 0)),
            pl.BlockSpec((1, N, C), lambda b: (b, 0, 0)),
        ],
        out_specs=pl.BlockSpec((1, COUT, N), lambda b: (b, 0, 0)),
        out_shape=jax.ShapeDtypeStruct((B, COUT, N), jnp.float32),
    )(Wx, Wj, bias.reshape(COUT, 1), x3, mr3)


def _block_diag(blocks):
    G, R, S = blocks.shape
    out = jnp.zeros((G * R, G * S), blocks.dtype)
    for g in range(G):
        out = out.at[g * R:(g + 1) * R, g * S:(g + 1) * S].set(blocks[g])
    return out


def kernel(x, edge_index, W, bias):
    B, C, N, _ = x.shape
    K = edge_index.shape[-1]
    COUT = W.shape[0]

    x3 = x[..., 0]
    xT = jnp.transpose(x3, (0, 2, 1)).reshape(B * N, C)
    ei = edge_index.astype(jnp.int32)
    base = (jnp.arange(B, dtype=jnp.int32) * N)[:, None, None]
    idx_j = (ei[0] + base).reshape(B * N * K)
    idx_i = (ei[1] + base).reshape(B * N * K)

    mr = _sc_maxrel(xT, idx_j, idx_i, B * N, C, K)

    Wg = W[:, :, 0, 0].reshape(_GROUPS, COUT // _GROUPS, (2 * C) // _GROUPS)
    Wx = _block_diag(Wg[:, :, 0::2])
    Wj = _block_diag(Wg[:, :, 1::2])

    out = _tc_conv(x3, mr.reshape(B, N, C), Wx, Wj, bias)
    return out[..., None]


# speedup vs baseline: 1.3433x; 1.3402x over previous
import functools

import numpy as np

import jax
import jax.numpy as jnp
from jax import lax
from jax.experimental import pallas as pl
from jax.experimental.pallas import tpu as pltpu
from jax.experimental.pallas import tpu_sc as plsc

_GROUPS = 4
_LANES = 16
_NC, _NS = 2, 16
_NW = _NC * _NS


def _sc_maxrel(xT, idx_j, idx_i, M, C, K):
    E = M * K
    EC = 128
    NPC = EC // K
    NCH = E // EC
    T = -(-NCH // _NW)
    T += T % 2
    NCHP = _NW * T
    Mp = NCHP * NPC

    ij2 = jnp.zeros((NCHP, EC), jnp.int32).at[:NCH].set(idx_j.reshape(NCH, EC))
    ii2 = jnp.zeros((NCHP, EC), jnp.int32).at[:NCH].set(idx_i.reshape(NCH, EC))

    mesh = plsc.VectorSubcoreMesh(core_axis_name="c", subcore_axis_name="s")

    @functools.partial(
        pl.kernel,
        mesh=mesh,
        compiler_params=pltpu.CompilerParams(use_tc_tiling_on_sc=False),
        out_type=jax.ShapeDtypeStruct((Mp, C), jnp.float32),
        scratch_types=[
            pltpu.VMEM((EC,), jnp.int32),
            pltpu.VMEM((EC,), jnp.int32),
            pltpu.VMEM((EC,), jnp.int32),
            pltpu.VMEM((EC,), jnp.int32),
            pltpu.VMEM((EC, C // 2), jnp.int32),
            pltpu.VMEM((EC, C // 2), jnp.int32),
            pltpu.VMEM((EC, C // 2), jnp.int32),
            pltpu.VMEM((EC, C // 2), jnp.int32),
            pltpu.VMEM((NPC, C), jnp.float32),
            pltpu.VMEM((NPC, C), jnp.float32),
            pltpu.SemaphoreType.DMA,
            pltpu.SemaphoreType.DMA,
            pltpu.SemaphoreType.DMA,
            pltpu.SemaphoreType.DMA,
            pltpu.SemaphoreType.DMA,
            pltpu.SemaphoreType.DMA,
        ],
    )
    def sc_kernel(xT_hbm, ij_hbm, ii_hbm, out_hbm,
                  ij0, ij1, ii0, ii1, rj0, rj1, ri0, ri1, ov0, ov1,
                  sj0, sj1, si0, si1, so0, so1):
        ij_v = (ij0, ij1)
        ii_v = (ii0, ii1)
        rj_v = (rj0, rj1)
        ri_v = (ri0, ri1)
        o_v = (ov0, ov1)
        semj = (sj0, sj1)
        semi = (si0, si1)
        semo = (so0, so1)
        wid = lax.axis_index("s") * _NC + lax.axis_index("c")

        def fetch(t, b):
            ch = wid + t * _NW
            pltpu.sync_copy(ij_hbm.at[ch], ij_v[b])
            pltpu.sync_copy(ii_hbm.at[ch], ii_v[b])
            pltpu.async_copy(xT_hbm.at[ij_v[b]], rj_v[b], semj[b])
            pltpu.async_copy(xT_hbm.at[ii_v[b]], ri_v[b], semi[b])

        for b in range(2):
            fetch(b, b)

        MASK = jnp.int32(-65536)

        def halves(v):
            lo = lax.bitcast_convert_type(v << 16, jnp.float32)
            hi = lax.bitcast_convert_type(v & MASK, jnp.float32)
            return lo, hi

        def compute(b):
            def node(n, c2):
                for cc in range(C // 2 // _LANES):
                    sl = pl.ds(cc * _LANES, _LANES)
                    je, jo = halves(rj_v[b][n * K, sl])
                    ie, io = halves(ri_v[b][n * K, sl])
                    acc_e = je - ie
                    acc_o = jo - io
                    for kk in range(1, K):
                        je, jo = halves(rj_v[b][n * K + kk, sl])
                        ie, io = halves(ri_v[b][n * K + kk, sl])
                        acc_e = jnp.maximum(acc_e, je - ie)
                        acc_o = jnp.maximum(acc_o, jo - io)
                    o_v[b][n, pl.ds(cc * 2 * _LANES, _LANES)] = acc_e
                    o_v[b][n, pl.ds(cc * 2 * _LANES + _LANES, _LANES)] = acc_o
                return c2
            lax.fori_loop(0, NPC, node, 0)

        def wait_gathers(b):
            pltpu.make_async_copy(xT_hbm.at[ij_v[b]], rj_v[b], semj[b]).wait()
            pltpu.make_async_copy(xT_hbm.at[ii_v[b]], ri_v[b], semi[b]).wait()

        def writeback(t, b):
            r0 = (wid + t * _NW) * NPC
            pltpu.sync_copy(o_v[b], out_hbm.at[pl.ds(r0, NPC)])

        for b in range(2):
            wait_gathers(b)
            compute(b)
            writeback(b, b)
            fetch(b + 2, b)

        def outer(g, carry):
            for b in range(2):
                t = g + b
                wait_gathers(b)
                compute(b)
                writeback(t, b)
                fetch(t + 2, b)
            return carry

        lax.fori_loop(0, (T - 4) // 2, lambda i, c: outer(2 + i * 2, c), 0)

        for b in range(2):
            wait_gathers(b)
            compute(b)
            writeback(T - 2 + b, b)

    out = sc_kernel(xT, ij2, ii2)
    return out[:M]


def _tc_body(wx_ref, wj_ref, b_ref, x_ref, mr_ref, o_ref):
    xb = x_ref[0]
    mr = mr_ref[0]
    acc = jnp.dot(wx_ref[...], xb, preferred_element_type=jnp.float32)
    acc = acc + lax.dot_general(
        wj_ref[...], mr, (((1,), (1,)), ((), ())),
        preferred_element_type=jnp.float32)
    o_ref[0] = jnp.maximum(acc + b_ref[...], 0.0)


def _tc_conv(x3, mr3, Wx, Wj, bias):
    B, C, N = x3.shape
    COUT = Wx.shape[0]
    grid = (B,)
    return pl.pallas_call(
        _tc_body,
        grid=grid,
        in_specs=[
            pl.BlockSpec((COUT, C), lambda b: (0, 0)),
            pl.BlockSpec((COUT, C), lambda b: (0, 0)),
            pl.BlockSpec((COUT, 1), lambda b: (0, 0)),
            pl.BlockSpec((1, C, N), lambda b: (b, 0, 0)),
            pl.BlockSpec((1, N, C), lambda b: (b, 0, 0)),
        ],
        out_specs=pl.BlockSpec((1, COUT, N), lambda b: (b, 0, 0)),
        out_shape=jax.ShapeDtypeStruct((B, COUT, N), jnp.float32),
    )(Wx, Wj, bias.reshape(COUT, 1), x3, mr3)


def _block_diag(blocks):
    G, R, S = blocks.shape
    out = jnp.zeros((G * R, G * S), blocks.dtype)
    for g in range(G):
        out = out.at[g * R:(g + 1) * R, g * S:(g + 1) * S].set(blocks[g])
    return out


def kernel(x, edge_index, W, bias):
    B, C, N, _ = x.shape
    K = edge_index.shape[-1]
    COUT = W.shape[0]

    x3 = x[..., 0]
    xT = jnp.transpose(x3, (0, 2, 1)).reshape(B * N, C)
    ei = edge_index.astype(jnp.int32)
    base = (jnp.arange(B, dtype=jnp.int32) * N)[:, None, None]
    idx_j = (ei[0] + base).reshape(B * N * K)
    idx_i = (ei[1] + base).reshape(B * N * K)

    xT32 = lax.bitcast_convert_type(
        xT.astype(jnp.bfloat16).reshape(B * N, C // 2, 2), jnp.int32)
    mr = _sc_maxrel(xT32, idx_j, idx_i, B * N, C, K)

    Wg = W[:, :, 0, 0].reshape(_GROUPS, COUT // _GROUPS, (2 * C) // _GROUPS)
    Wx = _block_diag(Wg[:, :, 0::2])
    Wj = _block_diag(Wg[:, :, 1::2])
    blk = np.arange(C).reshape(C // 32, 16, 2)
    order = np.concatenate([blk[:, :, 0], blk[:, :, 1]], axis=1).reshape(-1)
    Wj = Wj[:, order]

    out = _tc_conv(x3, mr.reshape(B, N, C), Wx, Wj, bias)
    return out[..., None]


# speedup vs baseline: 1.5946x; 1.1871x over previous
import functools

import numpy as np

import jax
import jax.numpy as jnp
from jax import lax
from jax.experimental import pallas as pl
from jax.experimental.pallas import tpu as pltpu
from jax.experimental.pallas import tpu_sc as plsc

_GROUPS = 4
_LANES = 16
_NC, _NS = 2, 16
_NW = _NC * _NS


def _sc_maxrel(xT, idx_j, idx_i, M, C, K):
    E = M * K
    EC = 128
    NPC = EC // K
    NCH = E // EC
    T = -(-NCH // _NW)
    NCHP = _NW * T
    Mp = NCHP * NPC
    CW = C // 2

    ij2 = jnp.zeros((NCHP, EC), jnp.int32).at[:NCH].set(idx_j.reshape(NCH, EC))
    ii2 = jnp.zeros((NCHP, EC), jnp.int32).at[:NCH].set(idx_i.reshape(NCH, EC))
    icat = jnp.stack([ij2, ii2], axis=1)

    mesh = plsc.VectorSubcoreMesh(core_axis_name="c", subcore_axis_name="s")

    @functools.partial(
        pl.kernel,
        mesh=mesh,
        compiler_params=pltpu.CompilerParams(use_tc_tiling_on_sc=False),
        out_type=jax.ShapeDtypeStruct((Mp, C), jnp.float32),
        scratch_types=[
            pltpu.VMEM((2, EC), jnp.int32),
            pltpu.VMEM((EC, CW), jnp.int32),
            pltpu.VMEM((EC, CW), jnp.int32),
            pltpu.VMEM((NPC, C), jnp.float32),
            pltpu.SemaphoreType.DMA,
            pltpu.SemaphoreType.DMA,
        ],
    )
    def sc_kernel(xT_hbm, ic_hbm, out_hbm, idx_v, rj_v, ri_v, o_v, semj, semi):
        wid = lax.axis_index("s") * _NC + lax.axis_index("c")

        MASK = jnp.int32(-65536)

        def halves(v):
            lo = lax.bitcast_convert_type(v << 16, jnp.float32)
            hi = lax.bitcast_convert_type(v & MASK, jnp.float32)
            return lo, hi

        def compute():
            def node(n, c2):
                for cc in range(CW // _LANES):
                    sl = pl.ds(cc * _LANES, _LANES)
                    je, jo = halves(rj_v[n * K, sl])
                    ie, io = halves(ri_v[n * K, sl])
                    acc_e = je - ie
                    acc_o = jo - io
                    for kk in range(1, K):
                        je, jo = halves(rj_v[n * K + kk, sl])
                        ie, io = halves(ri_v[n * K + kk, sl])
                        acc_e = jnp.maximum(acc_e, je - ie)
                        acc_o = jnp.maximum(acc_o, jo - io)
                    o_v[n, pl.ds(cc * 2 * _LANES, _LANES)] = acc_e
                    o_v[n, pl.ds(cc * 2 * _LANES + _LANES, _LANES)] = acc_o
                return c2
            lax.fori_loop(0, NPC, node, 0)

        def body(t, carry):
            ch = wid + t * _NW
            pltpu.sync_copy(ic_hbm.at[ch], idx_v)
            cj = pltpu.async_copy(xT_hbm.at[idx_v.at[0]], rj_v, semj)
            ci = pltpu.async_copy(xT_hbm.at[idx_v.at[1]], ri_v, semi)
            cj.wait()
            ci.wait()
            compute()
            pltpu.sync_copy(o_v, out_hbm.at[pl.ds(ch * NPC, NPC)])
            return carry

        lax.fori_loop(0, T, body, 0)

    out = sc_kernel(xT, icat)
    return out[:M]


def _tc_body(wx_ref, wj_ref, b_ref, x_ref, mr_ref, o_ref):
    xb = x_ref[0]
    mr = mr_ref[0]
    acc = jnp.dot(wx_ref[...], xb, preferred_element_type=jnp.float32)
    acc = acc + lax.dot_general(
        wj_ref[...], mr, (((1,), (1,)), ((), ())),
        preferred_element_type=jnp.float32)
    o_ref[0] = jnp.maximum(acc + b_ref[...], 0.0)


def _tc_conv(x3, mr3, Wx, Wj, bias):
    B, C, N = x3.shape
    COUT = Wx.shape[0]
    grid = (B,)
    return pl.pallas_call(
        _tc_body,
        grid=grid,
        in_specs=[
            pl.BlockSpec((COUT, C), lambda b: (0, 0)),
            pl.BlockSpec((COUT, C), lambda b: (0, 0)),
            pl.BlockSpec((COUT, 1), lambda b: (0, 0)),
            pl.BlockSpec((1, C, N), lambda b: (b, 0, 0)),
            pl.BlockSpec((1, N, C), lambda b: (b, 0, 0)),
        ],
        out_specs=pl.BlockSpec((1, COUT, N), lambda b: (b, 0, 0)),
        out_shape=jax.ShapeDtypeStruct((B, COUT, N), jnp.float32),
    )(Wx, Wj, bias.reshape(COUT, 1), x3, mr3)


def _block_diag(blocks):
    G, R, S = blocks.shape
    out = jnp.zeros((G * R, G * S), blocks.dtype)
    for g in range(G):
        out = out.at[g * R:(g + 1) * R, g * S:(g + 1) * S].set(blocks[g])
    return out


def kernel(x, edge_index, W, bias):
    B, C, N, _ = x.shape
    K = edge_index.shape[-1]
    COUT = W.shape[0]

    x3 = x[..., 0]
    xT = jnp.transpose(x3, (0, 2, 1)).reshape(B * N, C)
    ei = edge_index.astype(jnp.int32)
    base = (jnp.arange(B, dtype=jnp.int32) * N)[:, None, None]
    idx_j = (ei[0] + base).reshape(B * N * K)
    idx_i = (ei[1] + base).reshape(B * N * K)

    xT32 = lax.bitcast_convert_type(
        xT.astype(jnp.bfloat16).reshape(B * N, C // 2, 2), jnp.int32)
    mr = _sc_maxrel(xT32, idx_j, idx_i, B * N, C, K)

    Wg = W[:, :, 0, 0].reshape(_GROUPS, COUT // _GROUPS, (2 * C) // _GROUPS)
    Wx = _block_diag(Wg[:, :, 0::2])
    Wj = _block_diag(Wg[:, :, 1::2])
    blk = np.arange(C).reshape(C // 32, 16, 2)
    order = np.concatenate([blk[:, :, 0], blk[:, :, 1]], axis=1).reshape(-1)
    Wj = Wj[:, order]

    out = _tc_conv(x3, mr.reshape(B, N, C), Wx, Wj, bias)
    return out[..., None]
